# baseline (speedup 1.0000x reference)
import functools

import jax
import jax.numpy as jnp
from jax import lax
from jax.experimental import pallas as pl
from jax.experimental.pallas import tpu as pltpu
from jax.experimental.pallas import tpu_sc as plsc

NC = 2
NS = 16
NW = NC * NS
C = 128


@functools.cache
def _sc_segsum(n, d, g0, g1):
    assert g0 % 8 == 0 and g1 % 8 == 0
    rpt = (n // NS) // 8 * 8
    tail = n - NS * rpt
    assert tail % 8 == 0
    zrpt = -(-(n + 8) // (NS * 8)) * 8
    n_acc = NS * zrpt
    ncores = NC
    mesh = plsc.VectorSubcoreMesh(core_axis_name="c", subcore_axis_name="s",
                                  num_cores=ncores)

    @functools.partial(
        pl.kernel,
        out_type=jax.ShapeDtypeStruct((ncores, n, d), jnp.float32),
        mesh=mesh,
        scratch_types=[
            pltpu.VMEM_SHARED((n_acc, d), jnp.float32),
            pltpu.VMEM((8, C), jnp.int32),
            pltpu.VMEM((8, C), jnp.int32),
            pltpu.VMEM((C, d), jnp.float32),
            pltpu.VMEM((C, d), jnp.float32),
            pltpu.SemaphoreType.DMA,
            pltpu.SemaphoreType.DMA,
        ],
    )
    def seg(x_hbm, srcm, dstm, zeros_hbm, out_hbm, acc, src_v, dst_v,
            rows_a, rows_b, gsem_a, gsem_b):
        c = lax.axis_index("c")
        s = lax.axis_index("s")
        pltpu.sync_copy(zeros_hbm.at[pl.ds(s * zrpt, zrpt)],
                        acc.at[pl.ds(s * zrpt, zrpt)])
        plsc.subcore_barrier()

        def blk_body(base):
            pltpu.sync_copy(srcm.at[pl.ds(base, 8)], src_v)
            pltpu.sync_copy(dstm.at[pl.ds(base, 8)], dst_v)
            pltpu.async_copy(x_hbm.at[src_v.at[0]], rows_a, gsem_a)
            for j in range(0, 8, 2):
                pltpu.make_async_copy(x_hbm.at[src_v.at[j]], rows_a, gsem_a).wait()
                pltpu.async_copy(x_hbm.at[src_v.at[j + 1]], rows_b, gsem_b)
                pltpu.sync_copy(rows_a, acc.at[dst_v.at[j]], add=True)
                pltpu.make_async_copy(x_hbm.at[src_v.at[j + 1]], rows_b,
                                      gsem_b).wait()
                if j + 2 < 8:
                    pltpu.async_copy(x_hbm.at[src_v.at[j + 2]], rows_a, gsem_a)
                pltpu.sync_copy(rows_b, acc.at[dst_v.at[j + 1]], add=True)

        if g0:
            @pl.when(c == 0)
            def _():
                def blk(bi, carry):
                    blk_body(s * g0 + bi * 8)
                    return carry
                lax.fori_loop(0, g0 // 8, blk, 0)
        if g1:
            @pl.when(c == 1)
            def _():
                def blk(bi, carry):
                    blk_body(NS * g0 + s * g1 + bi * 8)
                    return carry
                lax.fori_loop(0, g1 // 8, blk, 0)
        plsc.subcore_barrier()
        pltpu.sync_copy(acc.at[pl.ds(s * rpt, rpt)],
                        out_hbm.at[c, pl.ds(s * rpt, rpt)])
        if tail:
            @pl.when(s == NS - 1)
            def _():
                pltpu.sync_copy(acc.at[pl.ds(NS * rpt, tail)],
                                out_hbm.at[c, pl.ds(NS * rpt, tail)])

    return seg


@functools.cache
def _tc_linear_relu(n, d, rblk, nparts):
    assert n % rblk == 0

    def body(*refs):
        parts, (w, b, out) = refs[:nparts], refs[nparts:]
        x = parts[0][...]
        for p in parts[1:]:
            x = x + p[...]
        out[...] = jnp.maximum(
            jnp.dot(x, w[...], preferred_element_type=jnp.float32) + b[...],
            0.0)

    return pl.pallas_call(
        body,
        grid=(n // rblk,),
        in_specs=[pl.BlockSpec((rblk, d), lambda i: (i, 0))] * nparts + [
            pl.BlockSpec((d, d), lambda i: (0, 0)),
            pl.BlockSpec((1, d), lambda i: (0, 0)),
        ],
        out_specs=pl.BlockSpec((rblk, d), lambda i: (i, 0)),
        out_shape=jax.ShapeDtypeStruct((n, d), jnp.float32),
    )


@functools.cache
def _tc_head(n, d, rblk, nparts):
    assert n % rblk == 0

    def body(*refs):
        parts, (w, b, hw, hb, out, acc) = refs[:nparts], refs[nparts:]
        i = pl.program_id(0)
        x = parts[0][...]
        for p in parts[1:]:
            x = x + p[...]
        h = jnp.maximum(
            jnp.dot(x, w[...], preferred_element_type=jnp.float32) + b[...],
            0.0)
        part = jnp.sum(h, axis=0, keepdims=True)

        @pl.when(i == 0)
        def _():
            acc[...] = part

        @pl.when(i > 0)
        def _():
            acc[...] = acc[...] + part

        @pl.when(i == pl.num_programs(0) - 1)
        def _():
            pooled = acc[...] * (1.0 / n)
            out[...] = jnp.sum(pooled * hw[...], axis=1, keepdims=True) + hb[...]

    return pl.pallas_call(
        body,
        grid=(n // rblk,),
        in_specs=[pl.BlockSpec((rblk, d), lambda i: (i, 0))] * nparts + [
            pl.BlockSpec((d, d), lambda i: (0, 0)),
            pl.BlockSpec((1, d), lambda i: (0, 0)),
            pl.BlockSpec((1, d), lambda i: (0, 0)),
            pl.BlockSpec((1, 1), lambda i: (0, 0)),
        ],
        out_specs=pl.BlockSpec((1, 1), lambda i: (0, 0)),
        out_shape=jax.ShapeDtypeStruct((1, 1), jnp.float32),
        scratch_shapes=[pltpu.VMEM((1, d), jnp.float32)],
    )


def kernel(node_features, edge_index, W1, b1, W2, b2, head_W, head_b):
    n, d = node_features.shape
    e = edge_index.shape[1]
    gsum = -(-e // (NS * C * 8)) * 8
    g0, g1 = gsum, 0
    e_pad = NS * C * gsum
    src = edge_index[0].astype(jnp.int32)
    dst = edge_index[1].astype(jnp.int32)
    pad = e_pad - e
    if pad:
        src = jnp.concatenate([src, jnp.zeros((pad,), jnp.int32)])
        dst = jnp.concatenate([dst, jnp.full((pad,), n, jnp.int32)])
    srcm = src.reshape(NS * gsum, C)
    dstm = dst.reshape(NS * gsum, C)
    zrpt = -(-(n + 8) // (NS * 8)) * 8
    zeros = jnp.zeros((NS * zrpt, d), jnp.float32)

    ncores = NC
    seg = _sc_segsum(n, d, g0, g1)
    lin = _tc_linear_relu(n, d, 1000, ncores)
    head = _tc_head(n, d, 1000, ncores)

    parts1 = seg(node_features, srcm, dstm, zeros)
    h1 = lin(*(parts1[i] for i in range(ncores)), W1, b1.reshape(1, d))
    parts2 = seg(h1, srcm, dstm, zeros)
    out = head(*(parts2[i] for i in range(ncores)), W2, b2.reshape(1, d),
               head_W.reshape(1, d), head_b.reshape(1, 1))
    return jnp.squeeze(out)


# speedup vs baseline: 2.9912x; 2.9912x over previous
import functools

import jax
import jax.numpy as jnp
from jax import lax
from jax.experimental import pallas as pl
from jax.experimental.pallas import tpu as pltpu
from jax.experimental.pallas import tpu_sc as plsc

NC = 2
NS = 16
C = 128


@functools.cache
def _sc_segsum_halves(n, dh, g):
    assert g % 8 == 0 and n % 8 == 0
    rpt = (n // NS) // 8 * 8
    tail = n - NS * rpt
    assert tail % 8 == 0
    zrpt = -(-(n + 8) // (NS * 8)) * 8
    n_acc = NS * zrpt
    mesh = plsc.VectorSubcoreMesh(core_axis_name="c", subcore_axis_name="s",
                                  num_cores=NC)

    @functools.partial(
        pl.kernel,
        out_type=jax.ShapeDtypeStruct((NC, n, dh), jnp.float32),
        mesh=mesh,
        scratch_types=[
            pltpu.VMEM_SHARED((n, dh), jnp.float32),
            pltpu.VMEM_SHARED((n_acc, dh), jnp.float32),
            pltpu.VMEM((8, C), jnp.int32),
            pltpu.VMEM((8, C), jnp.int32),
            pltpu.VMEM((C, dh), jnp.float32),
            pltpu.VMEM((C, dh), jnp.float32),
            pltpu.SemaphoreType.DMA,
            pltpu.SemaphoreType.DMA,
        ],
    )
    def seg(xh_hbm, srcm, dstm, zeros_hbm, out_hbm, tab, acc, src_v, dst_v,
            rows_a, rows_b, gsem_a, gsem_b):
        c = lax.axis_index("c")
        s = lax.axis_index("s")
        pltpu.sync_copy(xh_hbm.at[c, pl.ds(s * rpt, rpt)],
                        tab.at[pl.ds(s * rpt, rpt)])
        if tail:
            @pl.when(s == NS - 1)
            def _():
                pltpu.sync_copy(xh_hbm.at[c, pl.ds(NS * rpt, tail)],
                                tab.at[pl.ds(NS * rpt, tail)])
        pltpu.sync_copy(zeros_hbm.at[pl.ds(s * zrpt, zrpt)],
                        acc.at[pl.ds(s * zrpt, zrpt)])
        plsc.subcore_barrier()

        def blk(bi, carry):
            base = s * g + bi * 8
            pltpu.sync_copy(srcm.at[pl.ds(base, 8)], src_v)
            pltpu.sync_copy(dstm.at[pl.ds(base, 8)], dst_v)
            pltpu.async_copy(tab.at[src_v.at[0]], rows_a, gsem_a)
            for j in range(0, 8, 2):
                pltpu.make_async_copy(tab.at[src_v.at[j]], rows_a, gsem_a).wait()
                pltpu.async_copy(tab.at[src_v.at[j + 1]], rows_b, gsem_b)
                pltpu.sync_copy(rows_a, acc.at[dst_v.at[j]], add=True)
                pltpu.make_async_copy(tab.at[src_v.at[j + 1]], rows_b,
                                      gsem_b).wait()
                if j + 2 < 8:
                    pltpu.async_copy(tab.at[src_v.at[j + 2]], rows_a, gsem_a)
                pltpu.sync_copy(rows_b, acc.at[dst_v.at[j + 1]], add=True)
            return carry

        lax.fori_loop(0, g // 8, blk, 0)
        plsc.subcore_barrier()
        pltpu.sync_copy(acc.at[pl.ds(s * rpt, rpt)],
                        out_hbm.at[c, pl.ds(s * rpt, rpt)])
        if tail:
            @pl.when(s == NS - 1)
            def _():
                pltpu.sync_copy(acc.at[pl.ds(NS * rpt, tail)],
                                out_hbm.at[c, pl.ds(NS * rpt, tail)])

    return seg


@functools.cache
def _tc_linear_relu(n, d, rblk):
    assert n % rblk == 0
    dh = d // 2

    def body(p, w, b, out):
        x = p[...]
        h = jnp.dot(x[0], w[0:dh, :], preferred_element_type=jnp.float32)
        h = h + jnp.dot(x[1], w[dh:d, :], preferred_element_type=jnp.float32)
        h = jnp.maximum(h + b[...], 0.0)
        out[...] = jnp.stack([h[:, 0:dh], h[:, dh:d]])

    return pl.pallas_call(
        body,
        grid=(n // rblk,),
        in_specs=[
            pl.BlockSpec((2, rblk, dh), lambda i: (0, i, 0)),
            pl.BlockSpec((d, d), lambda i: (0, 0)),
            pl.BlockSpec((1, d), lambda i: (0, 0)),
        ],
        out_specs=pl.BlockSpec((2, rblk, dh), lambda i: (0, i, 0)),
        out_shape=jax.ShapeDtypeStruct((2, n, dh), jnp.float32),
    )


@functools.cache
def _tc_head(n, d, rblk):
    assert n % rblk == 0
    dh = d // 2

    def body(p, w, b, hw, hb, out, acc):
        i = pl.program_id(0)
        x = p[...]
        h = jnp.dot(x[0], w[0:dh, :], preferred_element_type=jnp.float32)
        h = h + jnp.dot(x[1], w[dh:d, :], preferred_element_type=jnp.float32)
        h = jnp.maximum(h + b[...], 0.0)
        part = jnp.sum(h, axis=0, keepdims=True)

        @pl.when(i == 0)
        def _():
            acc[...] = part

        @pl.when(i > 0)
        def _():
            acc[...] = acc[...] + part

        @pl.when(i == pl.num_programs(0) - 1)
        def _():
            pooled = acc[...] * (1.0 / n)
            out[...] = jnp.sum(pooled * hw[...], axis=1, keepdims=True) + hb[...]

    return pl.pallas_call(
        body,
        grid=(n // rblk,),
        in_specs=[
            pl.BlockSpec((2, rblk, dh), lambda i: (0, i, 0)),
            pl.BlockSpec((d, d), lambda i: (0, 0)),
            pl.BlockSpec((1, d), lambda i: (0, 0)),
            pl.BlockSpec((1, d), lambda i: (0, 0)),
            pl.BlockSpec((1, 1), lambda i: (0, 0)),
        ],
        out_specs=pl.BlockSpec((1, 1), lambda i: (0, 0)),
        out_shape=jax.ShapeDtypeStruct((1, 1), jnp.float32),
        scratch_shapes=[pltpu.VMEM((1, d), jnp.float32)],
    )


def kernel(node_features, edge_index, W1, b1, W2, b2, head_W, head_b):
    n, d = node_features.shape
    dh = d // 2
    e = edge_index.shape[1]
    g = -(-e // (NS * C * 8)) * 8
    e_pad = NS * C * g
    src = edge_index[0].astype(jnp.int32)
    dst = edge_index[1].astype(jnp.int32)
    pad = e_pad - e
    if pad:
        src = jnp.concatenate([src, jnp.zeros((pad,), jnp.int32)])
        dst = jnp.concatenate([dst, jnp.full((pad,), n, jnp.int32)])
    srcm = src.reshape(NS * g, C)
    dstm = dst.reshape(NS * g, C)
    zrpt = -(-(n + 8) // (NS * 8)) * 8
    zeros = jnp.zeros((NS * zrpt, dh), jnp.float32)

    seg = _sc_segsum_halves(n, dh, g)
    lin = _tc_linear_relu(n, d, 1000)
    head = _tc_head(n, d, 1000)

    x2 = jnp.stack([node_features[:, 0:dh], node_features[:, dh:d]])
    parts1 = seg(x2, srcm, dstm, zeros)
    h1 = lin(parts1, W1, b1.reshape(1, d))
    parts2 = seg(h1, srcm, dstm, zeros)
    out = head(parts2, W2, b2.reshape(1, d),
               head_W.reshape(1, d), head_b.reshape(1, 1))
    return jnp.squeeze(out)


# speedup vs baseline: 3.2886x; 1.0994x over previous
import functools

import jax
import jax.numpy as jnp
from jax import lax
from jax.experimental import pallas as pl
from jax.experimental.pallas import tpu as pltpu
from jax.experimental.pallas import tpu_sc as plsc

NC = 2
NS = 16
C = 128


@functools.cache
def _sc_segsum_halves(n, dh, g):
    assert g % 8 == 0 and n % 8 == 0
    rpt = (n // NS) // 8 * 8
    tail = n - NS * rpt
    assert tail % 8 == 0
    zrpt = -(-(n + 8) // (NS * 8)) * 8
    n_acc = NS * zrpt
    mesh = plsc.VectorSubcoreMesh(core_axis_name="c", subcore_axis_name="s",
                                  num_cores=NC)

    @functools.partial(
        pl.kernel,
        out_type=jax.ShapeDtypeStruct((NC, n, dh), jnp.float32),
        mesh=mesh,
        scratch_types=[
            pltpu.VMEM_SHARED((n, dh), jnp.float32),
            pltpu.VMEM_SHARED((n_acc, dh), jnp.float32),
            pltpu.VMEM((2, 8, C), jnp.int32),
            pltpu.VMEM((2, 8, C), jnp.int32),
            pltpu.VMEM((C, dh), jnp.float32),
            pltpu.VMEM((C, dh), jnp.float32),
            pltpu.SemaphoreType.DMA,
            pltpu.SemaphoreType.DMA,
            pltpu.SemaphoreType.DMA,
            pltpu.SemaphoreType.DMA,
        ],
    )
    def seg(xh_hbm, srcm, dstm, zeros_hbm, out_hbm, tab, acc, src_v, dst_v,
            rows_a, rows_b, gsem_a, gsem_b, isem_a, isem_b):
        c = lax.axis_index("c")
        s = lax.axis_index("s")
        pltpu.sync_copy(xh_hbm.at[c, pl.ds(s * rpt, rpt)],
                        tab.at[pl.ds(s * rpt, rpt)])
        if tail:
            @pl.when(s == NS - 1)
            def _():
                pltpu.sync_copy(xh_hbm.at[c, pl.ds(NS * rpt, tail)],
                                tab.at[pl.ds(NS * rpt, tail)])
        pltpu.sync_copy(zeros_hbm.at[pl.ds(s * zrpt, zrpt)],
                        acc.at[pl.ds(s * zrpt, zrpt)])
        plsc.subcore_barrier()

        nblk = g // 8
        pltpu.async_copy(srcm.at[pl.ds(s * g, 8)], src_v.at[0], isem_a)
        pltpu.async_copy(dstm.at[pl.ds(s * g, 8)], dst_v.at[0], isem_b)

        def blk(bi, carry):
            pb = lax.rem(bi, 2)
            sv = src_v.at[pb]
            dv = dst_v.at[pb]
            pltpu.make_async_copy(srcm.at[pl.ds(s * g, 8)], sv, isem_a).wait()
            pltpu.make_async_copy(dstm.at[pl.ds(s * g, 8)], dv, isem_b).wait()

            @pl.when(bi + 1 < nblk)
            def _():
                base_n = s * g + (bi + 1) * 8
                pltpu.async_copy(srcm.at[pl.ds(base_n, 8)],
                                 src_v.at[1 - pb], isem_a)
                pltpu.async_copy(dstm.at[pl.ds(base_n, 8)],
                                 dst_v.at[1 - pb], isem_b)

            pltpu.async_copy(tab.at[sv.at[0]], rows_a, gsem_a)
            for j in range(0, 8, 2):
                pltpu.make_async_copy(tab.at[sv.at[j]], rows_a, gsem_a).wait()
                pltpu.async_copy(tab.at[sv.at[j + 1]], rows_b, gsem_b)
                pltpu.sync_copy(rows_a, acc.at[dv.at[j]], add=True)
                pltpu.make_async_copy(tab.at[sv.at[j + 1]], rows_b,
                                      gsem_b).wait()
                if j + 2 < 8:
                    pltpu.async_copy(tab.at[sv.at[j + 2]], rows_a, gsem_a)
                pltpu.sync_copy(rows_b, acc.at[dv.at[j + 1]], add=True)
            return carry

        lax.fori_loop(0, nblk, blk, 0)
        plsc.subcore_barrier()
        pltpu.sync_copy(acc.at[pl.ds(s * rpt, rpt)],
                        out_hbm.at[c, pl.ds(s * rpt, rpt)])
        if tail:
            @pl.when(s == NS - 1)
            def _():
                pltpu.sync_copy(acc.at[pl.ds(NS * rpt, tail)],
                                out_hbm.at[c, pl.ds(NS * rpt, tail)])

    return seg


@functools.cache
def _tc_linear_relu(n, d, rblk):
    assert n % rblk == 0
    dh = d // 2

    def body(p, w, b, out):
        x = p[...]
        h = jnp.dot(x[0], w[0:dh, :], preferred_element_type=jnp.float32)
        h = h + jnp.dot(x[1], w[dh:d, :], preferred_element_type=jnp.float32)
        h = jnp.maximum(h + b[...], 0.0)
        out[...] = jnp.stack([h[:, 0:dh], h[:, dh:d]])

    return pl.pallas_call(
        body,
        grid=(n // rblk,),
        in_specs=[
            pl.BlockSpec((2, rblk, dh), lambda i: (0, i, 0)),
            pl.BlockSpec((d, d), lambda i: (0, 0)),
            pl.BlockSpec((1, d), lambda i: (0, 0)),
        ],
        out_specs=pl.BlockSpec((2, rblk, dh), lambda i: (0, i, 0)),
        out_shape=jax.ShapeDtypeStruct((2, n, dh), jnp.float32),
    )


@functools.cache
def _tc_head(n, d, rblk):
    assert n % rblk == 0
    dh = d // 2

    def body(p, w, b, hw, hb, out, acc):
        i = pl.program_id(0)
        x = p[...]
        h = jnp.dot(x[0], w[0:dh, :], preferred_element_type=jnp.float32)
        h = h + jnp.dot(x[1], w[dh:d, :], preferred_element_type=jnp.float32)
        h = jnp.maximum(h + b[...], 0.0)
        part = jnp.sum(h, axis=0, keepdims=True)

        @pl.when(i == 0)
        def _():
            acc[...] = part

        @pl.when(i > 0)
        def _():
            acc[...] = acc[...] + part

        @pl.when(i == pl.num_programs(0) - 1)
        def _():
            pooled = acc[...] * (1.0 / n)
            out[...] = jnp.sum(pooled * hw[...], axis=1, keepdims=True) + hb[...]

    return pl.pallas_call(
        body,
        grid=(n // rblk,),
        in_specs=[
            pl.BlockSpec((2, rblk, dh), lambda i: (0, i, 0)),
            pl.BlockSpec((d, d), lambda i: (0, 0)),
            pl.BlockSpec((1, d), lambda i: (0, 0)),
            pl.BlockSpec((1, d), lambda i: (0, 0)),
            pl.BlockSpec((1, 1), lambda i: (0, 0)),
        ],
        out_specs=pl.BlockSpec((1, 1), lambda i: (0, 0)),
        out_shape=jax.ShapeDtypeStruct((1, 1), jnp.float32),
        scratch_shapes=[pltpu.VMEM((1, d), jnp.float32)],
    )


def kernel(node_features, edge_index, W1, b1, W2, b2, head_W, head_b):
    n, d = node_features.shape
    dh = d // 2
    e = edge_index.shape[1]
    g = -(-e // (NS * C * 8)) * 8
    e_pad = NS * C * g
    src = edge_index[0].astype(jnp.int32)
    dst = edge_index[1].astype(jnp.int32)
    pad = e_pad - e
    if pad:
        src = jnp.concatenate([src, jnp.zeros((pad,), jnp.int32)])
        dst = jnp.concatenate([dst, jnp.full((pad,), n, jnp.int32)])
    srcm = src.reshape(NS * g, C)
    dstm = dst.reshape(NS * g, C)
    zrpt = -(-(n + 8) // (NS * 8)) * 8
    zeros = jnp.zeros((NS * zrpt, dh), jnp.float32)

    seg = _sc_segsum_halves(n, dh, g)
    lin = _tc_linear_relu(n, d, 1000)
    head = _tc_head(n, d, 1000)

    x2 = jnp.stack([node_features[:, 0:dh], node_features[:, dh:d]])
    parts1 = seg(x2, srcm, dstm, zeros)
    h1 = lin(parts1, W1, b1.reshape(1, d))
    parts2 = seg(h1, srcm, dstm, zeros)
    out = head(parts2, W2, b2.reshape(1, d),
               head_W.reshape(1, d), head_b.reshape(1, 1))
    return jnp.squeeze(out)
